# baseline (speedup 1.0000x reference)
import functools

import jax
import jax.numpy as jnp
from jax import lax
from jax.experimental import pallas as pl
from jax.experimental.pallas import tpu as pltpu
from jax.experimental.pallas import tpu_sc as plsc

N = 10000
E = 320000
NP = 10240
NC = 2
NS = 16
NW = NC * NS
CHUNK = 128
G = 5
NGRP = 16
CPT = NGRP * G
EPAD = NW * CPT * CHUNK
ECH = EPAD // CHUNK
ROWS_PER_TILE = NP // NS

_MESH = plsc.VectorSubcoreMesh(
    core_axis_name="c", subcore_axis_name="s", num_cores=NC, num_subcores=NS
)
_SC_PARAMS = pltpu.CompilerParams(use_tc_tiling_on_sc=False)


def _deg_body(col_hbm, ew_hbm, out_hbm, cidx, ewb, zbuf, acc_sh, isem0, isem1):
    c = lax.axis_index("c")
    s = lax.axis_index("s")
    w = c * NS + s
    isems = (isem0, isem1)

    for i in range(CHUNK // 16):
        zbuf[pl.ds(i * 16, 16)] = jnp.zeros((16,), jnp.float32)
    def _zero(i, carry):
        pltpu.sync_copy(zbuf, acc_sh.at[pl.ds(s * ROWS_PER_TILE + i * CHUNK, CHUNK)])
        return carry
    lax.fori_loop(0, ROWS_PER_TILE // CHUNK, _zero, 0)
    plsc.subcore_barrier()

    def issue_idx(t, b):
        r0 = (w * NGRP + t) * G
        pltpu.async_copy(col_hbm.at[pl.ds(r0, G)], cidx.at[b], isems[b])
        pltpu.async_copy(ew_hbm.at[pl.ds(r0, G)], ewb.at[b], isems[b])

    def wait_idx(t, b):
        r0 = (w * NGRP + t) * G
        pltpu.make_async_copy(col_hbm.at[pl.ds(r0, G)], cidx.at[b], isems[b]).wait()
        pltpu.make_async_copy(ew_hbm.at[pl.ds(r0, G)], ewb.at[b], isems[b]).wait()

    issue_idx(0, 0)
    issue_idx(1, 1)

    def loop_body(i, carry):
        for k in range(2):
            t = 2 * i + k
            b = k
            wait_idx(t, b)
            for j in range(G):
                pltpu.sync_copy(ewb.at[b, j], acc_sh.at[cidx.at[b, j]], add=True)
            @pl.when(t + 2 < NGRP)
            def _():
                issue_idx(t + 2, b)
        return carry
    lax.fori_loop(0, NGRP // 2, loop_body, 0)
    plsc.subcore_barrier()

    pltpu.sync_copy(
        acc_sh.at[pl.ds(s * ROWS_PER_TILE, ROWS_PER_TILE)],
        out_hbm.at[c, pl.ds(s * ROWS_PER_TILE, ROWS_PER_TILE)],
    )


_deg_call = functools.partial(
    pl.kernel,
    out_type=jax.ShapeDtypeStruct((NC, NP), jnp.float32),
    mesh=_MESH,
    scratch_types=[
        pltpu.VMEM((2, G, CHUNK), jnp.int32),
        pltpu.VMEM((2, G, CHUNK), jnp.float32),
        pltpu.VMEM((CHUNK,), jnp.float32),
        pltpu.VMEM_SHARED((NP,), jnp.float32),
        pltpu.SemaphoreType.DMA,
        pltpu.SemaphoreType.DMA,
    ],
    compiler_params=_SC_PARAMS,
)(_deg_body)


def _make_agg_body(D):
    def body(hp_hbm, row_hbm, col_hbm, ew_hbm, out_hbm,
             ridx, cidx, ewb, rows, zbuf, acc_sh,
             isem0, isem1, isem2, isem3, gsem0, gsem1, ssem0, ssem1):
        c = lax.axis_index("c")
        s = lax.axis_index("s")
        w = c * NS + s
        isems = (isem0, isem1, isem2, isem3)
        gsems = (gsem0, gsem1)
        ssems = (ssem0, ssem1)

        GE = G * CHUNK

        def issue_idx(t, bi):
            r0 = (w * NGRP + t) * GE
            pltpu.async_copy(row_hbm.at[pl.ds(r0, GE)], ridx.at[bi], isems[bi])
            pltpu.async_copy(col_hbm.at[pl.ds(r0, GE)], cidx.at[bi], isems[bi])
            pltpu.async_copy(ew_hbm.at[pl.ds(r0, GE)], ewb.at[bi], isems[bi])

        def wait_idx(t, bi):
            r0 = (w * NGRP + t) * GE
            pltpu.make_async_copy(row_hbm.at[pl.ds(r0, GE)], ridx.at[bi], isems[bi]).wait()
            pltpu.make_async_copy(col_hbm.at[pl.ds(r0, GE)], cidx.at[bi], isems[bi]).wait()
            pltpu.make_async_copy(ew_hbm.at[pl.ds(r0, GE)], ewb.at[bi], isems[bi]).wait()

        def fire_gathers(bi, br):
            pltpu.async_copy(hp_hbm.at[ridx.at[bi]], rows.at[br], gsems[br])

        def drain_gathers(bi, br):
            pltpu.make_async_copy(
                hp_hbm.at[ridx.at[bi]], rows.at[br], gsems[br]).wait()

        def scale(br, bi):
            def sbody(jj, carry):
                ew16 = ewb[bi, pl.ds(jj * 16, 16)]
                for k in range(16):
                    e = jj * 16 + k
                    sc = ew16[k]
                    for f in range(D // 16):
                        rows[br, e, pl.ds(f * 16, 16)] = (
                            rows[br, e, pl.ds(f * 16, 16)] * sc)
                return carry
            lax.fori_loop(0, GE // 16, sbody, 0)

        def fire_scatter(br, bi):
            pltpu.async_copy(
                rows.at[br], acc_sh.at[cidx.at[bi]], ssems[br], add=True)

        def wait_scatter(br, bi):
            pltpu.make_async_copy(
                rows.at[br], acc_sh.at[cidx.at[bi]], ssems[br]).wait()

        issue_idx(0, 0)
        issue_idx(1, 1)
        def _zfill(i, carry):
            for f in range(D // 16):
                zbuf[i, pl.ds(f * 16, 16)] = jnp.zeros((16,), jnp.float32)
            return carry
        lax.fori_loop(0, CHUNK, _zfill, 0)
        def _zero(i, carry):
            pltpu.sync_copy(
                zbuf, acc_sh.at[pl.ds(s * ROWS_PER_TILE + i * CHUNK, CHUNK)])
            return carry
        lax.fori_loop(0, ROWS_PER_TILE // CHUNK, _zero, 0)
        plsc.subcore_barrier()
        wait_idx(0, 0)
        fire_gathers(0, 0)

        def loop_body(i, carry):
            for k in range(4):
                t = 4 * i + k
                br = k % 2
                br1 = (k + 1) % 2
                bi = k
                bi1 = (k + 1) % 4
                @pl.when(t + 1 < NGRP)
                def _():
                    wait_idx(t + 1, bi1)
                    @pl.when(t >= 1)
                    def _():
                        wait_scatter(br1, (k + 3) % 4)
                    fire_gathers(bi1, br1)
                drain_gathers(bi, br)
                scale(br, bi)
                fire_scatter(br, bi)
                @pl.when(t + 2 < NGRP)
                def _():
                    issue_idx(t + 2, (k + 2) % 4)
            return carry
        lax.fori_loop(0, NGRP // 4, loop_body, 0)
        wait_scatter(0, (NGRP - 2) % 4)
        wait_scatter(1, (NGRP - 1) % 4)
        plsc.subcore_barrier()

        def _wb(i, carry):
            r0 = s * ROWS_PER_TILE + i * CHUNK
            pltpu.sync_copy(
                acc_sh.at[pl.ds(r0, CHUNK)], out_hbm.at[c, pl.ds(r0, CHUNK)])
            return carry
        lax.fori_loop(0, ROWS_PER_TILE // CHUNK, _wb, 0)
    return body


def _make_agg_call(D):
    return functools.partial(
        pl.kernel,
        out_type=jax.ShapeDtypeStruct((NC, NP, D), jnp.float32),
        mesh=_MESH,
        scratch_types=[
            pltpu.VMEM((4, G * CHUNK), jnp.int32),
            pltpu.VMEM((4, G * CHUNK), jnp.int32),
            pltpu.VMEM((4, G * CHUNK), jnp.float32),
            pltpu.VMEM((2, G * CHUNK, D), jnp.float32),
            pltpu.VMEM((CHUNK, D), jnp.float32),
            pltpu.VMEM_SHARED((NP, D), jnp.float32),
            pltpu.SemaphoreType.DMA,
            pltpu.SemaphoreType.DMA,
            pltpu.SemaphoreType.DMA,
            pltpu.SemaphoreType.DMA,
            pltpu.SemaphoreType.DMA,
            pltpu.SemaphoreType.DMA,
            pltpu.SemaphoreType.DMA,
            pltpu.SemaphoreType.DMA,
        ],
        compiler_params=_SC_PARAMS,
    )(_make_agg_body(D))


_agg32 = _make_agg_call(32)
_agg16 = _make_agg_call(16)


def _tc_first_body(degp_ref, feat_ref, w1_ref, dis_ref, hp_ref):
    deg = degp_ref[0] + degp_ref[1] + 1.0
    dis = jnp.where(deg > 0, lax.rsqrt(jnp.maximum(deg, 1e-12)), 0.0)
    dis_ref[...] = dis
    h = jnp.dot(feat_ref[...], w1_ref[...], preferred_element_type=jnp.float32)
    hp_ref[...] = h * dis[:, None]


def _tc_first(degp, featp, w1p):
    return pl.pallas_call(
        _tc_first_body,
        out_shape=(
            jax.ShapeDtypeStruct((NP,), jnp.float32),
            jax.ShapeDtypeStruct((NP, w1p.shape[1]), jnp.float32),
        ),
    )(degp, featp, w1p)


def _tc_mid_body(p_ref, hp_ref, dis_ref, b_ref, w_ref, hpn_ref):
    acc = p_ref[0] + p_ref[1] + hp_ref[...]
    dis = dis_ref[...]
    x = jnp.maximum(acc * dis[:, None] + b_ref[...], 0.0)
    hpn_ref[...] = (
        jnp.dot(x, w_ref[...], preferred_element_type=jnp.float32)
        * dis[:, None]
    )


def _tc_mid(p, hp, dis, bp, wp):
    return pl.pallas_call(
        _tc_mid_body,
        out_shape=jax.ShapeDtypeStruct((NP, wp.shape[1]), jnp.float32),
    )(p, hp, dis, bp, wp)


def _tc_last_body(p_ref, hp_ref, dis_ref, b_ref, out_ref):
    acc = p_ref[0] + p_ref[1] + hp_ref[...]
    x = acc * dis_ref[...][:, None] + b_ref[...]
    cols = x.shape[1]
    colid = lax.broadcasted_iota(jnp.int32, (NP, cols), 1)
    xm = jnp.where(colid < 10, x, -3.0e38)
    m = jnp.max(xm, axis=1, keepdims=True)
    ex = jnp.where(colid < 10, jnp.exp(x - m), 0.0)
    lse = jnp.log(jnp.sum(ex, axis=1, keepdims=True))
    out_ref[...] = x - m - lse


def _tc_last(p, hp, dis, bp):
    return pl.pallas_call(
        _tc_last_body,
        out_shape=jax.ShapeDtypeStruct((NP, hp.shape[1]), jnp.float32),
    )(p, hp, dis, bp)


def kernel(features, edges, weights, W1, b1, W2, b2, W3, b3, W4, b4):
    row = edges[0]
    col = edges[1]
    pad_e = EPAD - E
    rowp = jnp.concatenate([row, jnp.zeros((pad_e,), row.dtype)])
    colp = jnp.concatenate([col, jnp.zeros((pad_e,), col.dtype)])
    ewp = jnp.concatenate([weights, jnp.zeros((pad_e,), weights.dtype)])
    row2d = rowp.reshape(ECH, CHUNK)
    col2d = colp.reshape(ECH, CHUNK)
    ew2d = ewp.reshape(ECH, CHUNK)
    featp = jnp.pad(features, ((0, NP - N), (0, 0)))
    W1p = jnp.pad(W1, ((0, 0), (0, 12)))
    W2p = jnp.pad(W2, ((0, 12), (0, 12)))
    W3p = jnp.pad(W3, ((0, 12), (0, 6)))
    W4p = jnp.pad(W4, ((0, 6), (0, 6)))
    b1p = jnp.pad(b1, (0, 12))
    b2p = jnp.pad(b2, (0, 12))
    b3p = jnp.pad(b3, (0, 6))
    b4p = jnp.pad(b4, (0, 6))

    degp = _deg_call(col2d, ew2d)
    dis, hp = _tc_first(degp, featp, W1p)
    for agg, bp, wp in (
        (_agg32, b1p, W2p),
        (_agg32, b2p, W2p),
        (_agg32, b2p, W3p),
        (_agg16, b3p, W4p),
    ):
        p = agg(hp, rowp, colp, ewp)
        hp = _tc_mid(p, hp, dis, bp, wp)
    p = _agg16(hp, rowp, colp, ewp)
    out = _tc_last(p, hp, dis, b4p)
    return out[:N, :10]


# speedup vs baseline: 1.7765x; 1.7765x over previous
import functools

import jax
import jax.numpy as jnp
from jax import lax
from jax.experimental import pallas as pl
from jax.experimental.pallas import tpu as pltpu
from jax.experimental.pallas import tpu_sc as plsc

N = 10000
E = 320000
NP = 10240
NC = 2
NS = 16
NW = NC * NS
CHUNK = 128
G = 5
NGRP = 16
CPT = NGRP * G
EPAD = NW * CPT * CHUNK
ECH = EPAD // CHUNK
ROWS_PER_TILE = NP // NS

_MESH = plsc.VectorSubcoreMesh(
    core_axis_name="c", subcore_axis_name="s", num_cores=NC, num_subcores=NS
)
_SC_PARAMS = pltpu.CompilerParams(use_tc_tiling_on_sc=False)


def _deg_body(col_hbm, ew_hbm, out_hbm, cidx, ewb, zbuf, acc_sh, isem0, isem1):
    c = lax.axis_index("c")
    s = lax.axis_index("s")
    w = c * NS + s
    isems = (isem0, isem1)

    for i in range(CHUNK // 16):
        zbuf[pl.ds(i * 16, 16)] = jnp.zeros((16,), jnp.float32)
    def _zero(i, carry):
        pltpu.sync_copy(zbuf, acc_sh.at[pl.ds(s * ROWS_PER_TILE + i * CHUNK, CHUNK)])
        return carry
    lax.fori_loop(0, ROWS_PER_TILE // CHUNK, _zero, 0)
    plsc.subcore_barrier()

    def issue_idx(t, b):
        r0 = (w * NGRP + t) * G
        pltpu.async_copy(col_hbm.at[pl.ds(r0, G)], cidx.at[b], isems[b])
        pltpu.async_copy(ew_hbm.at[pl.ds(r0, G)], ewb.at[b], isems[b])

    def wait_idx(t, b):
        r0 = (w * NGRP + t) * G
        pltpu.make_async_copy(col_hbm.at[pl.ds(r0, G)], cidx.at[b], isems[b]).wait()
        pltpu.make_async_copy(ew_hbm.at[pl.ds(r0, G)], ewb.at[b], isems[b]).wait()

    issue_idx(0, 0)
    issue_idx(1, 1)

    def loop_body(i, carry):
        for k in range(2):
            t = 2 * i + k
            b = k
            wait_idx(t, b)
            for j in range(G):
                pltpu.sync_copy(ewb.at[b, j], acc_sh.at[cidx.at[b, j]], add=True)
            @pl.when(t + 2 < NGRP)
            def _():
                issue_idx(t + 2, b)
        return carry
    lax.fori_loop(0, NGRP // 2, loop_body, 0)
    plsc.subcore_barrier()

    pltpu.sync_copy(
        acc_sh.at[pl.ds(s * ROWS_PER_TILE, ROWS_PER_TILE)],
        out_hbm.at[c, pl.ds(s * ROWS_PER_TILE, ROWS_PER_TILE)],
    )


_deg_call = functools.partial(
    pl.kernel,
    out_type=jax.ShapeDtypeStruct((NC, NP), jnp.float32),
    mesh=_MESH,
    scratch_types=[
        pltpu.VMEM((2, G, CHUNK), jnp.int32),
        pltpu.VMEM((2, G, CHUNK), jnp.float32),
        pltpu.VMEM((CHUNK,), jnp.float32),
        pltpu.VMEM_SHARED((NP,), jnp.float32),
        pltpu.SemaphoreType.DMA,
        pltpu.SemaphoreType.DMA,
    ],
    compiler_params=_SC_PARAMS,
)(_deg_body)


def _make_agg_body(D):
    def body(hp_hbm, row_hbm, col_hbm, ew_hbm, out_hbm,
             ridx, cidx, ewb, rows, zbuf, hp_sh, acc_sh,
             isem0, isem1, isem2, isem3, gsem0, gsem1, ssem0, ssem1):
        c = lax.axis_index("c")
        s = lax.axis_index("s")
        w = c * NS + s
        isems = (isem0, isem1, isem2, isem3)
        gsems = (gsem0, gsem1)
        ssems = (ssem0, ssem1)

        GE = G * CHUNK

        def issue_idx(t, bi):
            r0 = (w * NGRP + t) * GE
            pltpu.async_copy(row_hbm.at[pl.ds(r0, GE)], ridx.at[bi], isems[bi])
            pltpu.async_copy(col_hbm.at[pl.ds(r0, GE)], cidx.at[bi], isems[bi])
            pltpu.async_copy(ew_hbm.at[pl.ds(r0, GE)], ewb.at[bi], isems[bi])

        def wait_idx(t, bi):
            r0 = (w * NGRP + t) * GE
            pltpu.make_async_copy(row_hbm.at[pl.ds(r0, GE)], ridx.at[bi], isems[bi]).wait()
            pltpu.make_async_copy(col_hbm.at[pl.ds(r0, GE)], cidx.at[bi], isems[bi]).wait()
            pltpu.make_async_copy(ew_hbm.at[pl.ds(r0, GE)], ewb.at[bi], isems[bi]).wait()

        def fire_gathers(bi, br):
            pltpu.async_copy(hp_sh.at[ridx.at[bi]], rows.at[br], gsems[br])

        def drain_gathers(bi, br):
            pltpu.make_async_copy(
                hp_sh.at[ridx.at[bi]], rows.at[br], gsems[br]).wait()

        def scale(br, bi):
            def sbody(jj, carry):
                ew16 = ewb[bi, pl.ds(jj * 16, 16)]
                for k in range(16):
                    e = jj * 16 + k
                    sc = ew16[k]
                    for f in range(D // 16):
                        rows[br, e, pl.ds(f * 16, 16)] = (
                            rows[br, e, pl.ds(f * 16, 16)] * sc)
                return carry
            lax.fori_loop(0, GE // 16, sbody, 0)

        def fire_scatter(br, bi):
            pltpu.async_copy(
                rows.at[br], acc_sh.at[cidx.at[bi]], ssems[br], add=True)

        def wait_scatter(br, bi):
            pltpu.make_async_copy(
                rows.at[br], acc_sh.at[cidx.at[bi]], ssems[br]).wait()

        issue_idx(0, 0)
        issue_idx(1, 1)
        pltpu.sync_copy(
            hp_hbm.at[pl.ds(s * ROWS_PER_TILE, ROWS_PER_TILE)],
            hp_sh.at[pl.ds(s * ROWS_PER_TILE, ROWS_PER_TILE)],
        )
        def _zfill(i, carry):
            for f in range(D // 16):
                zbuf[i, pl.ds(f * 16, 16)] = jnp.zeros((16,), jnp.float32)
            return carry
        lax.fori_loop(0, CHUNK, _zfill, 0)
        def _zero(i, carry):
            pltpu.sync_copy(
                zbuf, acc_sh.at[pl.ds(s * ROWS_PER_TILE + i * CHUNK, CHUNK)])
            return carry
        lax.fori_loop(0, ROWS_PER_TILE // CHUNK, _zero, 0)
        plsc.subcore_barrier()
        wait_idx(0, 0)
        fire_gathers(0, 0)

        def loop_body(i, carry):
            for k in range(4):
                t = 4 * i + k
                br = k % 2
                br1 = (k + 1) % 2
                bi = k
                bi1 = (k + 1) % 4
                @pl.when(t + 1 < NGRP)
                def _():
                    wait_idx(t + 1, bi1)
                    @pl.when(t >= 1)
                    def _():
                        wait_scatter(br1, (k + 3) % 4)
                    fire_gathers(bi1, br1)
                drain_gathers(bi, br)
                scale(br, bi)
                fire_scatter(br, bi)
                @pl.when(t + 2 < NGRP)
                def _():
                    issue_idx(t + 2, (k + 2) % 4)
            return carry
        lax.fori_loop(0, NGRP // 4, loop_body, 0)
        wait_scatter(0, (NGRP - 2) % 4)
        wait_scatter(1, (NGRP - 1) % 4)
        plsc.subcore_barrier()

        def _wb(i, carry):
            r0 = s * ROWS_PER_TILE + i * CHUNK
            pltpu.sync_copy(
                acc_sh.at[pl.ds(r0, CHUNK)], out_hbm.at[c, pl.ds(r0, CHUNK)])
            return carry
        lax.fori_loop(0, ROWS_PER_TILE // CHUNK, _wb, 0)
    return body


def _make_agg_call(D):
    return functools.partial(
        pl.kernel,
        out_type=jax.ShapeDtypeStruct((NC, NP, D), jnp.float32),
        mesh=_MESH,
        scratch_types=[
            pltpu.VMEM((4, G * CHUNK), jnp.int32),
            pltpu.VMEM((4, G * CHUNK), jnp.int32),
            pltpu.VMEM((4, G * CHUNK), jnp.float32),
            pltpu.VMEM((2, G * CHUNK, D), jnp.float32),
            pltpu.VMEM((CHUNK, D), jnp.float32),
            pltpu.VMEM_SHARED((NP, D), jnp.float32),
            pltpu.VMEM_SHARED((NP, D), jnp.float32),
            pltpu.SemaphoreType.DMA,
            pltpu.SemaphoreType.DMA,
            pltpu.SemaphoreType.DMA,
            pltpu.SemaphoreType.DMA,
            pltpu.SemaphoreType.DMA,
            pltpu.SemaphoreType.DMA,
            pltpu.SemaphoreType.DMA,
            pltpu.SemaphoreType.DMA,
        ],
        compiler_params=_SC_PARAMS,
    )(_make_agg_body(D))


_agg32 = _make_agg_call(32)
_agg16 = _make_agg_call(16)


def _tc_first_body(degp_ref, feat_ref, w1_ref, dis_ref, hp_ref):
    deg = degp_ref[0] + degp_ref[1] + 1.0
    dis = jnp.where(deg > 0, lax.rsqrt(jnp.maximum(deg, 1e-12)), 0.0)
    dis_ref[...] = dis
    h = jnp.dot(feat_ref[...], w1_ref[...], preferred_element_type=jnp.float32)
    hp_ref[...] = h * dis[:, None]


def _tc_first(degp, featp, w1p):
    return pl.pallas_call(
        _tc_first_body,
        out_shape=(
            jax.ShapeDtypeStruct((NP,), jnp.float32),
            jax.ShapeDtypeStruct((NP, w1p.shape[1]), jnp.float32),
        ),
    )(degp, featp, w1p)


def _tc_mid_body(p_ref, hp_ref, dis_ref, b_ref, w_ref, hpn_ref):
    acc = p_ref[0] + p_ref[1] + hp_ref[...]
    dis = dis_ref[...]
    x = jnp.maximum(acc * dis[:, None] + b_ref[...], 0.0)
    hpn_ref[...] = (
        jnp.dot(x, w_ref[...], preferred_element_type=jnp.float32)
        * dis[:, None]
    )


def _tc_mid(p, hp, dis, bp, wp):
    return pl.pallas_call(
        _tc_mid_body,
        out_shape=jax.ShapeDtypeStruct((NP, wp.shape[1]), jnp.float32),
    )(p, hp, dis, bp, wp)


def _tc_last_body(p_ref, hp_ref, dis_ref, b_ref, out_ref):
    acc = p_ref[0] + p_ref[1] + hp_ref[...]
    x = acc * dis_ref[...][:, None] + b_ref[...]
    cols = x.shape[1]
    colid = lax.broadcasted_iota(jnp.int32, (NP, cols), 1)
    xm = jnp.where(colid < 10, x, -3.0e38)
    m = jnp.max(xm, axis=1, keepdims=True)
    ex = jnp.where(colid < 10, jnp.exp(x - m), 0.0)
    lse = jnp.log(jnp.sum(ex, axis=1, keepdims=True))
    out_ref[...] = x - m - lse


def _tc_last(p, hp, dis, bp):
    return pl.pallas_call(
        _tc_last_body,
        out_shape=jax.ShapeDtypeStruct((NP, hp.shape[1]), jnp.float32),
    )(p, hp, dis, bp)


def kernel(features, edges, weights, W1, b1, W2, b2, W3, b3, W4, b4):
    row = edges[0]
    col = edges[1]
    pad_e = EPAD - E
    rowp = jnp.concatenate([row, jnp.zeros((pad_e,), row.dtype)])
    colp = jnp.concatenate([col, jnp.zeros((pad_e,), col.dtype)])
    ewp = jnp.concatenate([weights, jnp.zeros((pad_e,), weights.dtype)])
    row2d = rowp.reshape(ECH, CHUNK)
    col2d = colp.reshape(ECH, CHUNK)
    ew2d = ewp.reshape(ECH, CHUNK)
    featp = jnp.pad(features, ((0, NP - N), (0, 0)))
    W1p = jnp.pad(W1, ((0, 0), (0, 12)))
    W2p = jnp.pad(W2, ((0, 12), (0, 12)))
    W3p = jnp.pad(W3, ((0, 12), (0, 6)))
    W4p = jnp.pad(W4, ((0, 6), (0, 6)))
    b1p = jnp.pad(b1, (0, 12))
    b2p = jnp.pad(b2, (0, 12))
    b3p = jnp.pad(b3, (0, 6))
    b4p = jnp.pad(b4, (0, 6))

    degp = _deg_call(col2d, ew2d)
    dis, hp = _tc_first(degp, featp, W1p)
    for agg, bp, wp in (
        (_agg32, b1p, W2p),
        (_agg32, b2p, W2p),
        (_agg32, b2p, W3p),
        (_agg16, b3p, W4p),
    ):
        p = agg(hp, rowp, colp, ewp)
        hp = _tc_mid(p, hp, dis, bp, wp)
    p = _agg16(hp, rowp, colp, ewp)
    out = _tc_last(p, hp, dis, b4p)
    return out[:N, :10]


# speedup vs baseline: 1.8152x; 1.0218x over previous
import functools

import jax
import jax.numpy as jnp
from jax import lax
from jax.experimental import pallas as pl
from jax.experimental.pallas import tpu as pltpu
from jax.experimental.pallas import tpu_sc as plsc

N = 10000
E = 320000
NP = 10240
NC = 2
NS = 16
NW = NC * NS
CHUNK = 128
G = 5
NGRP = 16
CPT = NGRP * G
EPAD = NW * CPT * CHUNK
ECH = EPAD // CHUNK
ROWS_PER_TILE = NP // NS

_MESH = plsc.VectorSubcoreMesh(
    core_axis_name="c", subcore_axis_name="s", num_cores=NC, num_subcores=NS
)
_SC_PARAMS = pltpu.CompilerParams(use_tc_tiling_on_sc=False)


def _deg_body(col_hbm, ew_hbm, out_hbm, cidx, ewb, zbuf, acc_sh, isem0, isem1):
    c = lax.axis_index("c")
    s = lax.axis_index("s")
    w = c * NS + s
    isems = (isem0, isem1)

    for i in range(CHUNK // 16):
        zbuf[pl.ds(i * 16, 16)] = jnp.zeros((16,), jnp.float32)
    def _zero(i, carry):
        pltpu.sync_copy(zbuf, acc_sh.at[pl.ds(s * ROWS_PER_TILE + i * CHUNK, CHUNK)])
        return carry
    lax.fori_loop(0, ROWS_PER_TILE // CHUNK, _zero, 0)
    plsc.subcore_barrier()

    def issue_idx(t, b):
        r0 = (w * NGRP + t) * G
        pltpu.async_copy(col_hbm.at[pl.ds(r0, G)], cidx.at[b], isems[b])
        pltpu.async_copy(ew_hbm.at[pl.ds(r0, G)], ewb.at[b], isems[b])

    def wait_idx(t, b):
        r0 = (w * NGRP + t) * G
        pltpu.make_async_copy(col_hbm.at[pl.ds(r0, G)], cidx.at[b], isems[b]).wait()
        pltpu.make_async_copy(ew_hbm.at[pl.ds(r0, G)], ewb.at[b], isems[b]).wait()

    issue_idx(0, 0)
    issue_idx(1, 1)

    def loop_body(i, carry):
        for k in range(2):
            t = 2 * i + k
            b = k
            wait_idx(t, b)
            for j in range(G):
                pltpu.sync_copy(ewb.at[b, j], acc_sh.at[cidx.at[b, j]], add=True)
            @pl.when(t + 2 < NGRP)
            def _():
                issue_idx(t + 2, b)
        return carry
    lax.fori_loop(0, NGRP // 2, loop_body, 0)
    plsc.subcore_barrier()

    pltpu.sync_copy(
        acc_sh.at[pl.ds(s * ROWS_PER_TILE, ROWS_PER_TILE)],
        out_hbm.at[c, pl.ds(s * ROWS_PER_TILE, ROWS_PER_TILE)],
    )


_deg_call = functools.partial(
    pl.kernel,
    out_type=jax.ShapeDtypeStruct((NC, NP), jnp.float32),
    mesh=_MESH,
    scratch_types=[
        pltpu.VMEM((2, G, CHUNK), jnp.int32),
        pltpu.VMEM((2, G, CHUNK), jnp.float32),
        pltpu.VMEM((CHUNK,), jnp.float32),
        pltpu.VMEM_SHARED((NP,), jnp.float32),
        pltpu.SemaphoreType.DMA,
        pltpu.SemaphoreType.DMA,
    ],
    compiler_params=_SC_PARAMS,
)(_deg_body)


def _make_agg_body(D):
    def body(hp_hbm, row_hbm, col_hbm, ew_hbm, out_hbm,
             ridx, cidx, ewb, rows, zbuf, hp_sh, acc_sh,
             isem0, isem1, isem2, isem3, gsem0, gsem1, ssem0, ssem1):
        c = lax.axis_index("c")
        s = lax.axis_index("s")
        w = c * NS + s
        isems = (isem0, isem1, isem2, isem3)
        gsems = (gsem0, gsem1)
        ssems = (ssem0, ssem1)

        GE = G * CHUNK

        def issue_idx(t, bi):
            r0 = (w * NGRP + t) * GE
            pltpu.async_copy(row_hbm.at[pl.ds(r0, GE)], ridx.at[bi], isems[bi])
            pltpu.async_copy(col_hbm.at[pl.ds(r0, GE)], cidx.at[bi], isems[bi])
            pltpu.async_copy(ew_hbm.at[pl.ds(r0, GE)], ewb.at[bi], isems[bi])

        def wait_idx(t, bi):
            r0 = (w * NGRP + t) * GE
            pltpu.make_async_copy(row_hbm.at[pl.ds(r0, GE)], ridx.at[bi], isems[bi]).wait()
            pltpu.make_async_copy(col_hbm.at[pl.ds(r0, GE)], cidx.at[bi], isems[bi]).wait()
            pltpu.make_async_copy(ew_hbm.at[pl.ds(r0, GE)], ewb.at[bi], isems[bi]).wait()

        def fire_gathers(bi, br):
            pltpu.async_copy(hp_sh.at[ridx.at[bi]], rows.at[br], gsems[br])

        def drain_gathers(bi, br):
            pltpu.make_async_copy(
                hp_sh.at[ridx.at[bi]], rows.at[br], gsems[br]).wait()

        def scale(br, bi):
            def sbody(jj, carry):
                ew16 = ewb[bi, pl.ds(jj * 16, 16)]
                for k in range(16):
                    e = jj * 16 + k
                    sc = ew16[k]
                    for f in range(D // 16):
                        rows[br, e, pl.ds(f * 16, 16)] = (
                            rows[br, e, pl.ds(f * 16, 16)] * sc)
                return carry
            lax.fori_loop(0, GE // 16, sbody, 0)

        def fire_scatter(br, bi):
            pltpu.async_copy(
                rows.at[br], acc_sh.at[cidx.at[bi]], ssems[br], add=True)

        def wait_scatter(br, bi):
            pltpu.make_async_copy(
                rows.at[br], acc_sh.at[cidx.at[bi]], ssems[br]).wait()

        issue_idx(0, 0)
        issue_idx(1, 1)
        pltpu.sync_copy(
            hp_hbm.at[pl.ds(s * ROWS_PER_TILE, ROWS_PER_TILE)],
            hp_sh.at[pl.ds(s * ROWS_PER_TILE, ROWS_PER_TILE)],
        )
        def _zfill(i, carry):
            for f in range(D // 16):
                zbuf[i, pl.ds(f * 16, 16)] = jnp.zeros((16,), jnp.float32)
            return carry
        lax.fori_loop(0, CHUNK, _zfill, 0)
        def _zero(i, carry):
            pltpu.sync_copy(
                zbuf, acc_sh.at[pl.ds(s * ROWS_PER_TILE + i * CHUNK, CHUNK)])
            return carry
        lax.fori_loop(0, ROWS_PER_TILE // CHUNK, _zero, 0)
        plsc.subcore_barrier()
        wait_idx(0, 0)
        fire_gathers(0, 0)

        def loop_body(i, carry):
            for k in range(4):
                t = 4 * i + k
                br = k % 2
                br1 = (k + 1) % 2
                bi = k
                bi1 = (k + 1) % 4
                @pl.when(t + 1 < NGRP)
                def _():
                    wait_idx(t + 1, bi1)
                    @pl.when(t >= 1)
                    def _():
                        wait_scatter(br1, (k + 3) % 4)
                    fire_gathers(bi1, br1)
                drain_gathers(bi, br)
                scale(br, bi)
                fire_scatter(br, bi)
                @pl.when(t + 2 < NGRP)
                def _():
                    issue_idx(t + 2, (k + 2) % 4)
            return carry
        lax.fori_loop(0, NGRP // 4, loop_body, 0)
        wait_scatter(0, (NGRP - 2) % 4)
        wait_scatter(1, (NGRP - 1) % 4)
        plsc.subcore_barrier()

        r0 = s * ROWS_PER_TILE
        pltpu.sync_copy(
            acc_sh.at[pl.ds(r0, ROWS_PER_TILE)],
            out_hbm.at[c, pl.ds(r0, ROWS_PER_TILE)])
    return body


def _make_agg_call(D):
    return functools.partial(
        pl.kernel,
        out_type=jax.ShapeDtypeStruct((NC, NP, D), jnp.float32),
        mesh=_MESH,
        scratch_types=[
            pltpu.VMEM((4, G * CHUNK), jnp.int32),
            pltpu.VMEM((4, G * CHUNK), jnp.int32),
            pltpu.VMEM((4, G * CHUNK), jnp.float32),
            pltpu.VMEM((2, G * CHUNK, D), jnp.float32),
            pltpu.VMEM((CHUNK, D), jnp.float32),
            pltpu.VMEM_SHARED((NP, D), jnp.float32),
            pltpu.VMEM_SHARED((NP, D), jnp.float32),
            pltpu.SemaphoreType.DMA,
            pltpu.SemaphoreType.DMA,
            pltpu.SemaphoreType.DMA,
            pltpu.SemaphoreType.DMA,
            pltpu.SemaphoreType.DMA,
            pltpu.SemaphoreType.DMA,
            pltpu.SemaphoreType.DMA,
            pltpu.SemaphoreType.DMA,
        ],
        compiler_params=_SC_PARAMS,
    )(_make_agg_body(D))


_agg32 = _make_agg_call(32)
_agg16 = _make_agg_call(16)


def _tc_first_body(degp_ref, feat_ref, w1_ref, dis_ref, hp_ref):
    deg = degp_ref[0] + degp_ref[1] + 1.0
    dis = jnp.where(deg > 0, lax.rsqrt(jnp.maximum(deg, 1e-12)), 0.0)
    dis_ref[...] = dis
    h = jnp.dot(feat_ref[...], w1_ref[...], preferred_element_type=jnp.float32)
    hp_ref[...] = h * dis[:, None]


def _tc_first(degp, featp, w1p):
    return pl.pallas_call(
        _tc_first_body,
        out_shape=(
            jax.ShapeDtypeStruct((NP,), jnp.float32),
            jax.ShapeDtypeStruct((NP, w1p.shape[1]), jnp.float32),
        ),
    )(degp, featp, w1p)


def _tc_mid_body(p_ref, hp_ref, dis_ref, b_ref, w_ref, hpn_ref):
    acc = p_ref[0] + p_ref[1] + hp_ref[...]
    dis = dis_ref[...]
    x = jnp.maximum(acc * dis[:, None] + b_ref[...], 0.0)
    hpn_ref[...] = (
        jnp.dot(x, w_ref[...], preferred_element_type=jnp.float32)
        * dis[:, None]
    )


def _tc_mid(p, hp, dis, bp, wp):
    return pl.pallas_call(
        _tc_mid_body,
        out_shape=jax.ShapeDtypeStruct((NP, wp.shape[1]), jnp.float32),
    )(p, hp, dis, bp, wp)


def _tc_last_body(p_ref, hp_ref, dis_ref, b_ref, out_ref):
    acc = p_ref[0] + p_ref[1] + hp_ref[...]
    x = acc * dis_ref[...][:, None] + b_ref[...]
    cols = x.shape[1]
    colid = lax.broadcasted_iota(jnp.int32, (NP, cols), 1)
    xm = jnp.where(colid < 10, x, -3.0e38)
    m = jnp.max(xm, axis=1, keepdims=True)
    ex = jnp.where(colid < 10, jnp.exp(x - m), 0.0)
    lse = jnp.log(jnp.sum(ex, axis=1, keepdims=True))
    out_ref[...] = x - m - lse


def _tc_last(p, hp, dis, bp):
    return pl.pallas_call(
        _tc_last_body,
        out_shape=jax.ShapeDtypeStruct((NP, hp.shape[1]), jnp.float32),
    )(p, hp, dis, bp)


def kernel(features, edges, weights, W1, b1, W2, b2, W3, b3, W4, b4):
    row = edges[0]
    col = edges[1]
    pad_e = EPAD - E
    rowp = jnp.concatenate([row, jnp.zeros((pad_e,), row.dtype)])
    colp = jnp.concatenate([col, jnp.zeros((pad_e,), col.dtype)])
    ewp = jnp.concatenate([weights, jnp.zeros((pad_e,), weights.dtype)])
    row2d = rowp.reshape(ECH, CHUNK)
    col2d = colp.reshape(ECH, CHUNK)
    ew2d = ewp.reshape(ECH, CHUNK)
    featp = jnp.pad(features, ((0, NP - N), (0, 0)))
    W1p = jnp.pad(W1, ((0, 0), (0, 12)))
    W2p = jnp.pad(W2, ((0, 12), (0, 12)))
    W3p = jnp.pad(W3, ((0, 12), (0, 6)))
    W4p = jnp.pad(W4, ((0, 6), (0, 6)))
    b1p = jnp.pad(b1, (0, 12))
    b2p = jnp.pad(b2, (0, 12))
    b3p = jnp.pad(b3, (0, 6))
    b4p = jnp.pad(b4, (0, 6))

    degp = _deg_call(col2d, ew2d)
    dis, hp = _tc_first(degp, featp, W1p)
    for agg, bp, wp in (
        (_agg32, b1p, W2p),
        (_agg32, b2p, W2p),
        (_agg32, b2p, W3p),
        (_agg16, b3p, W4p),
    ):
        p = agg(hp, rowp, colp, ewp)
        hp = _tc_mid(p, hp, dis, bp, wp)
    p = _agg16(hp, rowp, colp, ewp)
    out = _tc_last(p, hp, dis, b4p)
    return out[:N, :10]
